# baseline (speedup 1.0000x reference)
import jax
import jax.numpy as jnp
from jax import lax
from jax.experimental import pallas as pl
from jax.experimental.pallas import tpu as pltpu
from jax.experimental.pallas import tpu_sc as plsc

N = 10000
R = 8
E = 320000
EMB = 128
H = 16
C = 16

NC = 2
NS = 16
NW = NC * NS
EPW = E // NW
K = 128
NCH = 79
EPP = NCH * K
PAD = EPP - EPW
L = 16
GR = K // L
S_BASE = R * N

_mesh = plsc.VectorSubcoreMesh(core_axis_name="c", subcore_axis_name="s")

_sc_params = pltpu.CompilerParams(
    needs_layout_passes=False, use_tc_tiling_on_sc=False
)


def _xs_body(emb_ref, w_ref, b_ref, out_ref):
    out_ref[...] = (
        jnp.dot(emb_ref[...], w_ref[0], preferred_element_type=jnp.float32)
        + b_ref[0]
    )


def _mid_body(p0_ref, p1_ref, b_ref, out_ref):
    out_ref[...] = jnp.maximum(p0_ref[...] + p1_ref[...] + b_ref[...], 0.0)


def _final_body(q_ref, w2_ref, b_ref, out_ref):
    r = pl.program_id(0)

    @pl.when(r == 0)
    def _():
        out_ref[...] = jnp.broadcast_to(b_ref[...], (N, C))

    h2r = q_ref[0, 0] + q_ref[1, 0]
    out_ref[...] += jnp.dot(h2r, w2_ref[0], preferred_element_type=jnp.float32)


def _sc1_body(xs_hbm, ps_hbm, src_hbm, scidx_hbm, h2sc_hbm, rel_hbm, dst_hbm,
              values_hbm, h2idx_hbm, h1p0_hbm, h1p1_hbm,
              srcA, scidxA, h2scA, relA, dstA, vals_all, h2i_all,
              sidx_v, didx_v, xwidx_v, a_v, b_v, xw_v, msg_v,
              ps_v, zbuf_v, h1_sh, sem_gat0, sem_gat1):
    cid = lax.axis_index("c")
    sid = lax.axis_index("s")
    wid = cid * NS + sid
    zr = 1000
    sem_gat = (sem_gat0, sem_gat1)

    pltpu.sync_copy(src_hbm.at[wid], srcA)
    pltpu.sync_copy(scidx_hbm.at[wid], scidxA)
    pltpu.sync_copy(h2sc_hbm.at[wid], h2scA)
    pltpu.sync_copy(rel_hbm.at[wid], relA)
    pltpu.sync_copy(dst_hbm.at[wid], dstA)
    pltpu.sync_copy(ps_hbm, ps_v)

    @pl.loop(0, zr)
    def _(i):
        zbuf_v[i, :] = jnp.zeros((H,), jnp.float32)

    @pl.when(sid < N // zr)
    def _():
        pltpu.sync_copy(zbuf_v, h1_sh.at[pl.ds(sid * zr, zr)])

    @pl.when(sid == NS - 1)
    def _():
        pltpu.sync_copy(zbuf_v.at[pl.ds(0, 8)], h1_sh.at[pl.ds(N, 8)])

    plsc.subcore_barrier()

    iota = lax.broadcasted_iota(jnp.int32, (L,), 0)

    def gat_copies(b):
        return (
            pltpu.make_async_copy(xs_hbm.at[sidx_v.at[b]], a_v.at[b],
                                  sem_gat[b]),
            pltpu.make_async_copy(xs_hbm.at[didx_v.at[b]], b_v.at[b],
                                  sem_gat[b]),
            pltpu.make_async_copy(xs_hbm.at[xwidx_v.at[b]], xw_v.at[b],
                                  sem_gat[b]),
        )

    def prefetch(b, ci):
        for g in range(GR):
            sl = pl.ds(g * L, L)
            rl = relA[ci, sl]
            dl = dstA[ci, sl]
            xwidx_v[b, sl] = rl * N + dl
            sidx_v[b, sl] = srcA[ci, sl] + S_BASE
            didx_v[b, sl] = dl + S_BASE
        for c in gat_copies(b):
            c.start()

    def compute(b, ci):
        for c in gat_copies(b):
            c.wait()
        for g in range(GR):
            rows = iota + g * L
            rl = relA[ci, pl.ds(g * L, L)]
            acc = jnp.zeros((L,), jnp.float32)
            for h in range(H):
                hv = jnp.full((L,), h, jnp.int32)
                at = plsc.load_gather(a_v.at[b], [rows, hv])
                bt = plsc.load_gather(b_v.at[b], [rows, hv])
                pt = plsc.load_gather(ps_v, [rl, hv])
                acc = acc + at * pt * bt
            vals_all[ci, pl.ds(g * L, L)] = acc * 0.25
            h2i_all[ci, pl.ds(g * L, L)] = rl * N + h2scA[ci, pl.ds(g * L, L)]
            ce = jnp.full((L,), ci, jnp.int32)
            for j in range(L):
                e = g * L + j
                bc = plsc.load_gather(vals_all,
                                      [ce, jnp.full((L,), e, jnp.int32)])
                msg_v[e, :] = xw_v[b, e, :] * bc
        pltpu.sync_copy(msg_v, h1_sh.at[scidxA.at[ci]], add=True)

    prefetch(0, 0)

    @pl.loop(0, (NCH - 1) // 2)
    def _pair(k):
        ci = 2 * k
        prefetch(1, ci + 1)
        compute(0, ci)
        prefetch(0, ci + 2)
        compute(1, ci + 1)

    compute(0, NCH - 1)

    plsc.subcore_barrier()

    pltpu.sync_copy(vals_all, values_hbm.at[wid])
    pltpu.sync_copy(h2i_all, h2idx_hbm.at[wid])

    @pl.when((sid < N // zr) & (cid == 0))
    def _():
        pltpu.sync_copy(h1_sh.at[pl.ds(sid * zr, zr)],
                        h1p0_hbm.at[pl.ds(sid * zr, zr)])

    @pl.when((sid < N // zr) & (cid == 1))
    def _():
        pltpu.sync_copy(h1_sh.at[pl.ds(sid * zr, zr)],
                        h1p1_hbm.at[pl.ds(sid * zr, zr)])


def _sc2_body(h1_hbm, dst_hbm, h2idx_hbm, values_hbm,
              h2p_hbm,
              dstA, h2iA, valsA, hr_v, msg_v,
              zbuf_v, h2_sh, sem_gat0, sem_gat1):
    cid = lax.axis_index("c")
    sid = lax.axis_index("s")
    wid = cid * NS + sid
    zr = (R * N) // NS
    zb = zbuf_v.shape[0]
    sem_gat = (sem_gat0, sem_gat1)

    pltpu.sync_copy(dst_hbm.at[wid], dstA)
    pltpu.sync_copy(h2idx_hbm.at[wid], h2iA)
    pltpu.sync_copy(values_hbm.at[wid], valsA)

    @pl.loop(0, zb)
    def _(i):
        zbuf_v[i, :] = jnp.zeros((H,), jnp.float32)

    for j in range(zr // zb):
        pltpu.sync_copy(zbuf_v, h2_sh.at[pl.ds(sid * zr + j * zb, zb)])

    @pl.when(sid == NS - 1)
    def _():
        pltpu.sync_copy(zbuf_v.at[pl.ds(0, 8)], h2_sh.at[pl.ds(R * N, 8)])

    plsc.subcore_barrier()

    def gat_copy(b, ci):
        return pltpu.make_async_copy(h1_hbm.at[dstA.at[ci]], hr_v.at[b],
                                     sem_gat[b])

    def compute(b, ci):
        gat_copy(b, ci).wait()
        ce = jnp.full((L,), ci, jnp.int32)
        for e in range(K):
            bc = plsc.load_gather(valsA, [ce, jnp.full((L,), e, jnp.int32)])
            msg_v[e, :] = hr_v[b, e, :] * bc
        pltpu.sync_copy(msg_v, h2_sh.at[h2iA.at[ci]], add=True)

    gat_copy(0, 0).start()

    @pl.loop(0, (NCH - 1) // 2)
    def _pair(k):
        ci = 2 * k
        gat_copy(1, ci + 1).start()
        compute(0, ci)
        gat_copy(0, ci + 2).start()
        compute(1, ci + 1)

    compute(0, NCH - 1)

    plsc.subcore_barrier()
    pltpu.sync_copy(h2_sh.at[pl.ds(sid * zr, zr)],
                    h2p_hbm.at[cid, pl.ds(sid * zr, zr)])


def _pad_blocks(x, fill):
    x2 = x.reshape(NW, EPW)
    x2 = jnp.pad(x2, ((0, 0), (0, PAD)), constant_values=fill)
    return x2.reshape(NW, NCH, K)


def kernel(embeddings, weights1, weights2, bias1, bias2, sscore_w, sscore_b,
           pscore, src, rel, dst):
    f32 = jnp.float32
    i32 = jnp.int32
    src = src.astype(i32)
    rel = rel.astype(i32)
    dst = dst.astype(i32)
    src3 = _pad_blocks(src, 0)
    scidx3 = _pad_blocks(src, N)
    h2sc3 = _pad_blocks(src, R * N)
    rel3 = _pad_blocks(rel, 0)
    dst3 = _pad_blocks(dst, 0)
    bias1_2 = bias1.reshape(1, H)
    bias2_2 = bias2.reshape(1, C)

    w_all = jnp.concatenate([weights1, sscore_w.T[None]], axis=0)
    b_all = jnp.concatenate(
        [jnp.zeros((R, 1, H), f32), sscore_b.reshape(1, 1, H)], axis=0
    )

    xs_tab = pl.pallas_call(
        _xs_body,
        grid=(R + 1,),
        in_specs=[
            pl.BlockSpec((N, EMB), lambda r: (0, 0)),
            pl.BlockSpec((1, EMB, H), lambda r: (r, 0, 0)),
            pl.BlockSpec((1, 1, H), lambda r: (r, 0, 0)),
        ],
        out_specs=pl.BlockSpec((N, H), lambda r: (r, 0)),
        out_shape=jax.ShapeDtypeStruct(((R + 1) * N, H), f32),
    )(embeddings, w_all, b_all)

    sc1 = pl.kernel(
        _sc1_body,
        out_type=[
            jax.ShapeDtypeStruct((NW, NCH, K), f32),
            jax.ShapeDtypeStruct((NW, NCH, K), i32),
            jax.ShapeDtypeStruct((N, H), f32),
            jax.ShapeDtypeStruct((N, H), f32),
        ],
        mesh=_mesh,
        scratch_types=[
            pltpu.VMEM((NCH, K), i32),
            pltpu.VMEM((NCH, K), i32),
            pltpu.VMEM((NCH, K), i32),
            pltpu.VMEM((NCH, K), i32),
            pltpu.VMEM((NCH, K), i32),
            pltpu.VMEM((NCH, K), f32),
            pltpu.VMEM((NCH, K), i32),
            pltpu.VMEM((2, K), i32),
            pltpu.VMEM((2, K), i32),
            pltpu.VMEM((2, K), i32),
            pltpu.VMEM((2, K, H), f32),
            pltpu.VMEM((2, K, H), f32),
            pltpu.VMEM((2, K, H), f32),
            pltpu.VMEM((K, H), f32),
            pltpu.VMEM((R, H), f32),
            pltpu.VMEM((1000, H), f32),
            pltpu.VMEM_SHARED((N + 8, H), f32),
            pltpu.SemaphoreType.DMA,
            pltpu.SemaphoreType.DMA,
        ],
        compiler_params=_sc_params,
    )
    values, h2idx, h1p0, h1p1 = sc1(xs_tab, pscore, src3, scidx3, h2sc3, rel3, dst3)

    h1 = pl.pallas_call(
        _mid_body,
        out_shape=jax.ShapeDtypeStruct((N, H), f32),
    )(h1p0, h1p1, bias1_2)

    sc2 = pl.kernel(
        _sc2_body,
        out_type=jax.ShapeDtypeStruct((NC, R * N, H), f32),
        mesh=_mesh,
        scratch_types=[
            pltpu.VMEM((NCH, K), i32),
            pltpu.VMEM((NCH, K), i32),
            pltpu.VMEM((NCH, K), f32),
            pltpu.VMEM((2, K, H), f32),
            pltpu.VMEM((K, H), f32),
            pltpu.VMEM((250, H), f32),
            pltpu.VMEM_SHARED((R * N + 8, H), f32),
            pltpu.SemaphoreType.DMA,
            pltpu.SemaphoreType.DMA,
        ],
        compiler_params=_sc_params,
    )
    h2p = sc2(h1, dst3, h2idx, values)

    out = pl.pallas_call(
        _final_body,
        grid=(R,),
        in_specs=[
            pl.BlockSpec((NC, 1, N, H), lambda r: (0, r, 0, 0)),
            pl.BlockSpec((1, H, C), lambda r: (r, 0, 0)),
            pl.BlockSpec((1, C), lambda r: (0, 0)),
        ],
        out_specs=pl.BlockSpec((N, C), lambda r: (0, 0)),
        out_shape=jax.ShapeDtypeStruct((N, C), f32),
    )(h2p.reshape(NC, R, N, H), weights2, bias2_2)

    return out


# speedup vs baseline: 1.3303x; 1.3303x over previous
import jax
import jax.numpy as jnp
from jax import lax
from jax.experimental import pallas as pl
from jax.experimental.pallas import tpu as pltpu
from jax.experimental.pallas import tpu_sc as plsc

N = 10000
R = 8
E = 320000
EMB = 128
H = 16
C = 16

NC = 2
NS = 16
NW = NC * NS
EPW = E // NW
K = 80
NCH = 125
EPP = NCH * K
PAD = EPP - EPW
L = 16
GR = K // L
S_BASE = R * N

_mesh = plsc.VectorSubcoreMesh(core_axis_name="c", subcore_axis_name="s")

_sc_params = pltpu.CompilerParams(
    needs_layout_passes=False, use_tc_tiling_on_sc=False
)


def _xs_body(emb_ref, w_ref, b_ref, out_ref):
    out_ref[...] = (
        jnp.dot(emb_ref[...], w_ref[0], preferred_element_type=jnp.float32)
        + b_ref[0]
    )


def _mid_body(p0_ref, p1_ref, b_ref, out_ref):
    out_ref[...] = jnp.maximum(p0_ref[...] + p1_ref[...] + b_ref[...], 0.0)


def _final_body(q_ref, w2_ref, b_ref, out_ref):
    r = pl.program_id(0)

    @pl.when(r == 0)
    def _():
        out_ref[...] = jnp.broadcast_to(b_ref[...], (N, C))

    h2r = q_ref[0, 0] + q_ref[1, 0]
    out_ref[...] += jnp.dot(h2r, w2_ref[0], preferred_element_type=jnp.float32)


def _sc1_body(xs_hbm, ps_hbm, src_hbm, scidx_hbm, h2sc_hbm, rel_hbm, dst_hbm,
              values_hbm, h2idx_hbm, h1p0_hbm, h1p1_hbm,
              srcA, scidxA, h2scA, relA, dstA, vals_all, h2i_all,
              sidx_v, didx_v, xwidx_v, a_v, b_v, xw_v, msg_v,
              ps_v, zbuf_v, h1_sh, sem_gat0, sem_gat1):
    cid = lax.axis_index("c")
    sid = lax.axis_index("s")
    wid = cid * NS + sid
    zr = 1000
    sem_gat = (sem_gat0, sem_gat1)

    pltpu.sync_copy(src_hbm.at[wid], srcA)
    pltpu.sync_copy(scidx_hbm.at[wid], scidxA)
    pltpu.sync_copy(h2sc_hbm.at[wid], h2scA)
    pltpu.sync_copy(rel_hbm.at[wid], relA)
    pltpu.sync_copy(dst_hbm.at[wid], dstA)
    pltpu.sync_copy(ps_hbm, ps_v)

    @pl.loop(0, zr)
    def _(i):
        zbuf_v[i, :] = jnp.zeros((H,), jnp.float32)

    @pl.when(sid < N // zr)
    def _():
        pltpu.sync_copy(zbuf_v, h1_sh.at[pl.ds(sid * zr, zr)])

    @pl.when(sid == NS - 1)
    def _():
        pltpu.sync_copy(zbuf_v.at[pl.ds(0, 8)], h1_sh.at[pl.ds(N, 8)])

    plsc.subcore_barrier()

    iota = lax.broadcasted_iota(jnp.int32, (L,), 0)

    def gat_copies(b):
        return (
            pltpu.make_async_copy(xs_hbm.at[sidx_v.at[b]], a_v.at[b],
                                  sem_gat[b]),
            pltpu.make_async_copy(xs_hbm.at[didx_v.at[b]], b_v.at[b],
                                  sem_gat[b]),
            pltpu.make_async_copy(xs_hbm.at[xwidx_v.at[b]], xw_v.at[b],
                                  sem_gat[b]),
        )

    def prefetch(b, ci):
        for g in range(GR):
            sl = pl.ds(g * L, L)
            rl = relA[ci, sl]
            dl = dstA[ci, sl]
            xwidx_v[b, sl] = rl * N + dl
            sidx_v[b, sl] = srcA[ci, sl] + S_BASE
            didx_v[b, sl] = dl + S_BASE
        for c in gat_copies(b):
            c.start()

    def compute(b, ci):
        for c in gat_copies(b):
            c.wait()
        for g in range(GR):
            rows = iota + g * L
            rl = relA[ci, pl.ds(g * L, L)]
            acc = jnp.zeros((L,), jnp.float32)
            for h in range(H):
                hv = jnp.full((L,), h, jnp.int32)
                at = plsc.load_gather(a_v.at[b], [rows, hv])
                bt = plsc.load_gather(b_v.at[b], [rows, hv])
                pt = plsc.load_gather(ps_v, [rl, hv])
                acc = acc + at * pt * bt
            vals = acc * 0.25
            vals_all[ci, pl.ds(g * L, L)] = vals
            h2i_all[ci, pl.ds(g * L, L)] = rl * N + h2scA[ci, pl.ds(g * L, L)]
            for j in range(L):
                e = g * L + j
                msg_v[e, :] = xw_v[b, e, :] * vals[j]
        pltpu.sync_copy(msg_v, h1_sh.at[scidxA.at[ci]], add=True)

    prefetch(0, 0)

    @pl.loop(0, (NCH - 1) // 2)
    def _pair(k):
        ci = 2 * k
        prefetch(1, ci + 1)
        compute(0, ci)
        prefetch(0, ci + 2)
        compute(1, ci + 1)

    compute(0, NCH - 1)

    plsc.subcore_barrier()

    pltpu.sync_copy(vals_all, values_hbm.at[wid])
    pltpu.sync_copy(h2i_all, h2idx_hbm.at[wid])

    @pl.when((sid < N // zr) & (cid == 0))
    def _():
        pltpu.sync_copy(h1_sh.at[pl.ds(sid * zr, zr)],
                        h1p0_hbm.at[pl.ds(sid * zr, zr)])

    @pl.when((sid < N // zr) & (cid == 1))
    def _():
        pltpu.sync_copy(h1_sh.at[pl.ds(sid * zr, zr)],
                        h1p1_hbm.at[pl.ds(sid * zr, zr)])


def _sc2_body(h1_hbm, dst_hbm, h2idx_hbm, values_hbm,
              h2p_hbm,
              dstA, h2iA, valsA, hr_v, msg_v,
              zbuf_v, h2_sh, sem_gat0, sem_gat1):
    cid = lax.axis_index("c")
    sid = lax.axis_index("s")
    wid = cid * NS + sid
    zr = (R * N) // NS
    zb = zbuf_v.shape[0]
    sem_gat = (sem_gat0, sem_gat1)

    pltpu.sync_copy(dst_hbm.at[wid], dstA)
    pltpu.sync_copy(h2idx_hbm.at[wid], h2iA)
    pltpu.sync_copy(values_hbm.at[wid], valsA)

    @pl.loop(0, zb)
    def _(i):
        zbuf_v[i, :] = jnp.zeros((H,), jnp.float32)

    for j in range(zr // zb):
        pltpu.sync_copy(zbuf_v, h2_sh.at[pl.ds(sid * zr + j * zb, zb)])

    @pl.when(sid == NS - 1)
    def _():
        pltpu.sync_copy(zbuf_v.at[pl.ds(0, 8)], h2_sh.at[pl.ds(R * N, 8)])

    plsc.subcore_barrier()

    def gat_copy(b, ci):
        return pltpu.make_async_copy(h1_hbm.at[dstA.at[ci]], hr_v.at[b],
                                     sem_gat[b])

    def compute(b, ci):
        gat_copy(b, ci).wait()
        for g in range(GR):
            vals = valsA[ci, pl.ds(g * L, L)]
            for j in range(L):
                e = g * L + j
                msg_v[e, :] = hr_v[b, e, :] * vals[j]
        pltpu.sync_copy(msg_v, h2_sh.at[h2iA.at[ci]], add=True)

    gat_copy(0, 0).start()

    @pl.loop(0, (NCH - 1) // 2)
    def _pair(k):
        ci = 2 * k
        gat_copy(1, ci + 1).start()
        compute(0, ci)
        gat_copy(0, ci + 2).start()
        compute(1, ci + 1)

    compute(0, NCH - 1)

    plsc.subcore_barrier()
    pltpu.sync_copy(h2_sh.at[pl.ds(sid * zr, zr)],
                    h2p_hbm.at[cid, pl.ds(sid * zr, zr)])


def _pad_blocks(x, fill):
    x2 = x.reshape(NW, EPW)
    x2 = jnp.pad(x2, ((0, 0), (0, PAD)), constant_values=fill)
    return x2.reshape(NW, NCH, K)


def kernel(embeddings, weights1, weights2, bias1, bias2, sscore_w, sscore_b,
           pscore, src, rel, dst):
    f32 = jnp.float32
    i32 = jnp.int32
    src = src.astype(i32)
    rel = rel.astype(i32)
    dst = dst.astype(i32)
    src3 = _pad_blocks(src, 0)
    scidx3 = _pad_blocks(src, N)
    h2sc3 = _pad_blocks(src, R * N)
    rel3 = _pad_blocks(rel, 0)
    dst3 = _pad_blocks(dst, 0)
    bias1_2 = bias1.reshape(1, H)
    bias2_2 = bias2.reshape(1, C)

    w_all = jnp.concatenate([weights1, sscore_w.T[None]], axis=0)
    b_all = jnp.concatenate(
        [jnp.zeros((R, 1, H), f32), sscore_b.reshape(1, 1, H)], axis=0
    )

    xs_tab = pl.pallas_call(
        _xs_body,
        grid=(R + 1,),
        in_specs=[
            pl.BlockSpec((N, EMB), lambda r: (0, 0)),
            pl.BlockSpec((1, EMB, H), lambda r: (r, 0, 0)),
            pl.BlockSpec((1, 1, H), lambda r: (r, 0, 0)),
        ],
        out_specs=pl.BlockSpec((N, H), lambda r: (r, 0)),
        out_shape=jax.ShapeDtypeStruct(((R + 1) * N, H), f32),
    )(embeddings, w_all, b_all)

    sc1 = pl.kernel(
        _sc1_body,
        out_type=[
            jax.ShapeDtypeStruct((NW, NCH, K), f32),
            jax.ShapeDtypeStruct((NW, NCH, K), i32),
            jax.ShapeDtypeStruct((N, H), f32),
            jax.ShapeDtypeStruct((N, H), f32),
        ],
        mesh=_mesh,
        scratch_types=[
            pltpu.VMEM((NCH, K), i32),
            pltpu.VMEM((NCH, K), i32),
            pltpu.VMEM((NCH, K), i32),
            pltpu.VMEM((NCH, K), i32),
            pltpu.VMEM((NCH, K), i32),
            pltpu.VMEM((NCH, K), f32),
            pltpu.VMEM((NCH, K), i32),
            pltpu.VMEM((2, K), i32),
            pltpu.VMEM((2, K), i32),
            pltpu.VMEM((2, K), i32),
            pltpu.VMEM((2, K, H), f32),
            pltpu.VMEM((2, K, H), f32),
            pltpu.VMEM((2, K, H), f32),
            pltpu.VMEM((K, H), f32),
            pltpu.VMEM((R, H), f32),
            pltpu.VMEM((1000, H), f32),
            pltpu.VMEM_SHARED((N + 8, H), f32),
            pltpu.SemaphoreType.DMA,
            pltpu.SemaphoreType.DMA,
        ],
        compiler_params=_sc_params,
    )
    values, h2idx, h1p0, h1p1 = sc1(xs_tab, pscore, src3, scidx3, h2sc3, rel3, dst3)

    h1 = pl.pallas_call(
        _mid_body,
        out_shape=jax.ShapeDtypeStruct((N, H), f32),
    )(h1p0, h1p1, bias1_2)

    sc2 = pl.kernel(
        _sc2_body,
        out_type=jax.ShapeDtypeStruct((NC, R * N, H), f32),
        mesh=_mesh,
        scratch_types=[
            pltpu.VMEM((NCH, K), i32),
            pltpu.VMEM((NCH, K), i32),
            pltpu.VMEM((NCH, K), f32),
            pltpu.VMEM((2, K, H), f32),
            pltpu.VMEM((K, H), f32),
            pltpu.VMEM((250, H), f32),
            pltpu.VMEM_SHARED((R * N + 8, H), f32),
            pltpu.SemaphoreType.DMA,
            pltpu.SemaphoreType.DMA,
        ],
        compiler_params=_sc_params,
    )
    h2p = sc2(h1, dst3, h2idx, values)

    out = pl.pallas_call(
        _final_body,
        grid=(R,),
        in_specs=[
            pl.BlockSpec((NC, 1, N, H), lambda r: (0, r, 0, 0)),
            pl.BlockSpec((1, H, C), lambda r: (r, 0, 0)),
            pl.BlockSpec((1, C), lambda r: (0, 0)),
        ],
        out_specs=pl.BlockSpec((N, C), lambda r: (0, 0)),
        out_shape=jax.ShapeDtypeStruct((N, C), f32),
    )(h2p.reshape(NC, R, N, H), weights2, bias2_2)

    return out
